# baseline (speedup 1.0000x reference)
import functools

import jax
import jax.numpy as jnp
from jax import lax
from jax.experimental import pallas as pl
from jax.experimental.pallas import tpu as pltpu
from jax.experimental.pallas import tpu_sc as plsc

D = 128
L = 16
NC, NS = 2, 16
NW = NC * NS
B = 128


@functools.lru_cache(maxsize=None)
def _build(E):
    assert E % NW == 0
    epw = E // NW
    nchunk = -(-epw // B)
    assert nchunk % 2 == 1 and epw % 8 == 0

    mesh = plsc.VectorSubcoreMesh(core_axis_name="c", subcore_axis_name="s")

    @functools.partial(
        pl.kernel,
        out_type=jax.ShapeDtypeStruct((E,), jnp.float32),
        mesh=mesh,
        compiler_params=pltpu.CompilerParams(needs_layout_passes=False,
                                             use_tc_tiling_on_sc=False),
        scratch_types=[
            pltpu.VMEM((epw,), jnp.int32),
            pltpu.VMEM((epw,), jnp.int32),
            pltpu.VMEM((epw,), jnp.float32),
            pltpu.VMEM((B, D // 2), jnp.int32),
            pltpu.VMEM((B, D // 2), jnp.int32),
            pltpu.VMEM((B, D // 2), jnp.int32),
            pltpu.VMEM((B, D // 2), jnp.int32),
            pltpu.SemaphoreType.DMA,
            pltpu.SemaphoreType.DMA,
            pltpu.SemaphoreType.DMA,
            pltpu.SemaphoreType.DMA,
            pltpu.VMEM_SHARED((10000, D // 2), jnp.int32),
        ],
    )
    def scores_kernel(h_hbm, src_hbm, dst_hbm, out_hbm,
                      idx_s, idx_d, res, rs0, rd0, rs1, rd1,
                      sem_s0, sem_d0, sem_s1, sem_d1, h_spm):
        wid = lax.axis_index("s") * NC + lax.axis_index("c")
        base = wid * epw
        row_iota = lax.iota(jnp.int32, L)
        bufs = ((rs0, rd0, sem_s0, sem_d0), (rs1, rd1, sem_s1, sem_d1))

        @pl.when(lax.axis_index("s") == 0)
        def _():
            pltpu.sync_copy(h_hbm, h_spm)

        pltpu.async_copy(src_hbm.at[pl.ds(base, epw)], idx_s, sem_s0).wait()
        pltpu.async_copy(dst_hbm.at[pl.ds(base, epw)], idx_d, sem_d0).wait()
        plsc.subcore_barrier()

        def chunk_off(c):
            return jnp.minimum(c * B, epw - B)

        def gathers(b, c):
            rs, rd, sem_s, sem_d = bufs[b]
            off = chunk_off(c)
            cs = pltpu.make_async_copy(h_spm.at[idx_s.at[pl.ds(off, B)]],
                                       rs, sem_s)
            cd = pltpu.make_async_copy(h_hbm.at[idx_d.at[pl.ds(off, B)]],
                                       rd, sem_d)
            return cs, cd

        def issue(b, c):
            cs, cd = gathers(b, c)
            cs.start()
            cd.start()

        def wait(b, c):
            cs, cd = gathers(b, c)
            cs.wait()
            cd.wait()

        def compute(b, c):
            rs, rd, _, _ = bufs[b]
            off = chunk_off(c)

            def block(t, bcarry):
                i0 = t * L
                blk = jnp.zeros((L,), jnp.float32)
                for e in range(L):
                    i = i0 + e
                    accs = []
                    for j in range(4):
                        vs = plsc.bitcast(rs[i, pl.ds(L * j, L)], jnp.bfloat16)
                        vd = plsc.bitcast(rd[i, pl.ds(L * j, L)], jnp.bfloat16)
                        s0, s1 = plsc.unpack(vs, format=plsc.PackFormat.INTERLEAVED)
                        d0, d1 = plsc.unpack(vd, format=plsc.PackFormat.INTERLEAVED)
                        accs.append(s0 * d0 + s1 * d1)
                    acc = (accs[0] + accs[1]) + (accs[2] + accs[3])
                    blk = jnp.where(row_iota == e, jnp.sum(acc), blk)
                res[pl.ds(off + i0, L)] = blk
                return bcarry

            lax.fori_loop(0, B // L, block, 0)

        last = nchunk - 1
        issue(0, jnp.int32(0))
        issue(1, jnp.int32(1))

        def pair(i, carry):
            c0 = 2 * i
            wait(0, c0)
            compute(0, c0)
            issue(0, jnp.minimum(c0 + 2, last))
            c1 = c0 + 1
            wait(1, c1)
            compute(1, c1)
            issue(1, jnp.minimum(c1 + 2, last))
            return carry

        lax.fori_loop(0, (nchunk - 1) // 2, pair, 0)
        wait(0, jnp.int32(last))
        compute(0, jnp.int32(last))
        wait(1, jnp.int32(last))

        pltpu.sync_copy(res, out_hbm.at[pl.ds(base, epw)])

    return scores_kernel


def kernel(h, edge_index):
    src = edge_index[0].astype(jnp.int32)
    dst = edge_index[1].astype(jnp.int32)
    hb = h.astype(jnp.bfloat16)
    hb32 = jax.lax.bitcast_convert_type(
        hb.reshape(h.shape[0], h.shape[1] // 2, 2), jnp.int32)
    scores = _build(src.shape[0])(hb32, src, dst)
    return scores[:, None]


# speedup vs baseline: 1.0457x; 1.0457x over previous
import functools

import jax
import jax.numpy as jnp
from jax import lax
from jax.experimental import pallas as pl
from jax.experimental.pallas import tpu as pltpu
from jax.experimental.pallas import tpu_sc as plsc

D = 128
L = 16
NC, NS = 2, 16
NW = NC * NS
B = 128


@functools.lru_cache(maxsize=None)
def _build(E):
    assert E % NW == 0
    epw = E // NW
    nchunk = -(-epw // B)
    assert nchunk % 2 == 1 and epw % 8 == 0

    mesh = plsc.VectorSubcoreMesh(core_axis_name="c", subcore_axis_name="s")

    @functools.partial(
        pl.kernel,
        out_type=jax.ShapeDtypeStruct((E,), jnp.float32),
        mesh=mesh,
        compiler_params=pltpu.CompilerParams(needs_layout_passes=False,
                                             use_tc_tiling_on_sc=False),
        scratch_types=[
            pltpu.VMEM((epw,), jnp.int32),
            pltpu.VMEM((epw,), jnp.int32),
            pltpu.VMEM((epw,), jnp.float32),
            pltpu.VMEM((B, D // 2), jnp.int32),
            pltpu.VMEM((B, D // 2), jnp.int32),
            pltpu.VMEM((B, D // 2), jnp.int32),
            pltpu.VMEM((B, D // 2), jnp.int32),
            pltpu.SemaphoreType.DMA,
            pltpu.SemaphoreType.DMA,
            pltpu.SemaphoreType.DMA,
            pltpu.SemaphoreType.DMA,
            pltpu.VMEM_SHARED((10000, D // 2), jnp.int32),
        ],
    )
    def scores_kernel(h_hbm, src_hbm, dst_hbm, out_hbm,
                      idx_s, idx_d, res, rs0, rd0, rs1, rd1,
                      sem_s0, sem_d0, sem_s1, sem_d1, h_spm):
        wid = lax.axis_index("s") * NC + lax.axis_index("c")
        base = wid * epw
        row_iota = lax.iota(jnp.int32, L)
        bufs = ((rs0, rd0, sem_s0, sem_d0), (rs1, rd1, sem_s1, sem_d1))

        ci = pltpu.make_async_copy(src_hbm.at[pl.ds(base, epw)], idx_s, sem_s0)
        cj = pltpu.make_async_copy(dst_hbm.at[pl.ds(base, epw)], idx_d, sem_d0)
        ci.start()
        cj.start()
        n_nodes = h_hbm.shape[0]
        slab = n_nodes // NS
        s0 = lax.axis_index("s") * slab
        pltpu.sync_copy(h_hbm.at[pl.ds(s0, slab)], h_spm.at[pl.ds(s0, slab)])
        ci.wait()
        cj.wait()
        plsc.subcore_barrier()

        def chunk_off(c):
            return jnp.minimum(c * B, epw - B)

        def gathers(b, c):
            rs, rd, sem_s, sem_d = bufs[b]
            off = chunk_off(c)
            cs = pltpu.make_async_copy(h_spm.at[idx_s.at[pl.ds(off, B)]],
                                       rs, sem_s)
            cd = pltpu.make_async_copy(h_spm.at[idx_d.at[pl.ds(off, B)]],
                                       rd, sem_d)
            return cs, cd

        def issue(b, c):
            cs, cd = gathers(b, c)
            cs.start()
            cd.start()

        def wait(b, c):
            cs, cd = gathers(b, c)
            cs.wait()
            cd.wait()

        def compute(b, c):
            rs, rd, _, _ = bufs[b]
            off = chunk_off(c)

            def block(t, bcarry):
                i0 = t * L
                blk = jnp.zeros((L,), jnp.float32)
                for e in range(L):
                    i = i0 + e
                    accs = []
                    for j in range(4):
                        vs = plsc.bitcast(rs[i, pl.ds(L * j, L)], jnp.bfloat16)
                        vd = plsc.bitcast(rd[i, pl.ds(L * j, L)], jnp.bfloat16)
                        s0, s1 = plsc.unpack(vs, format=plsc.PackFormat.INTERLEAVED)
                        d0, d1 = plsc.unpack(vd, format=plsc.PackFormat.INTERLEAVED)
                        accs.append(s0 * d0 + s1 * d1)
                    acc = (accs[0] + accs[1]) + (accs[2] + accs[3])
                    blk = jnp.where(row_iota == e, jnp.sum(acc), blk)
                res[pl.ds(off + i0, L)] = blk
                return bcarry

            lax.fori_loop(0, B // L, block, 0)

        last = nchunk - 1
        issue(0, jnp.int32(0))
        issue(1, jnp.int32(1))

        def pair(i, carry):
            c0 = 2 * i
            wait(0, c0)
            compute(0, c0)
            issue(0, jnp.minimum(c0 + 2, last))
            c1 = c0 + 1
            wait(1, c1)
            compute(1, c1)
            issue(1, jnp.minimum(c1 + 2, last))
            return carry

        lax.fori_loop(0, (nchunk - 1) // 2, pair, 0)
        wait(0, jnp.int32(last))
        compute(0, jnp.int32(last))
        wait(1, jnp.int32(last))

        pltpu.sync_copy(res, out_hbm.at[pl.ds(base, epw)])

    return scores_kernel


def kernel(h, edge_index):
    src = edge_index[0].astype(jnp.int32)
    dst = edge_index[1].astype(jnp.int32)
    hb = h.astype(jnp.bfloat16)
    hb32 = jax.lax.bitcast_convert_type(
        hb.reshape(h.shape[0], h.shape[1] // 2, 2), jnp.int32)
    scores = _build(src.shape[0])(hb32, src, dst)
    return scores[:, None]
